# Initial kernel scaffold; baseline (speedup 1.0000x reference)
import jax
import jax.numpy as jnp
from jax.experimental import pallas as pl


def kernel(text, offsets, emb, W, b):
    raise NotImplementedError("write your pallas kernel here")


# speedup vs baseline: 13.3603x; 13.3603x over previous
import functools

import jax
import jax.numpy as jnp
from jax import lax
from jax.experimental import pallas as pl
from jax.experimental.pallas import tpu as pltpu
from jax.experimental.pallas import tpu_sc as plsc

B = 4096
SEQ = 200
D = 64
DP = 16
NC, NS = 2, 16
NW = NC * NS
BPW = B // NW
GRP = 8
NGRP = BPW // GRP
GROWS = GRP * SEQ


def _proj_body(emb_ref, wp_ref, p_ref):
    p_ref[...] = jnp.dot(
        emb_ref[...], wp_ref[...], preferred_element_type=jnp.float32
    ) * (1.0 / SEQ)


def _project(emb, wp):
    v = emb.shape[0]
    blk = 2000
    return pl.pallas_call(
        _proj_body,
        grid=(v // blk,),
        in_specs=[
            pl.BlockSpec((blk, D), lambda i: (i, 0)),
            pl.BlockSpec((D, DP), lambda i: (0, 0)),
        ],
        out_specs=pl.BlockSpec((blk, DP), lambda i: (i, 0)),
        out_shape=jax.ShapeDtypeStruct((v, DP), jnp.float32),
    )(emb, wp)


def _make_sc_kernel():
    mesh = plsc.VectorSubcoreMesh(core_axis_name="c", subcore_axis_name="s")

    @functools.partial(
        pl.kernel,
        mesh=mesh,
        compiler_params=pltpu.CompilerParams(use_tc_tiling_on_sc=False),
        out_type=jax.ShapeDtypeStruct((B, DP), jnp.float32),
        scratch_types=[
            pltpu.VMEM((BPW * SEQ,), jnp.int32),
            pltpu.VMEM((GROWS, DP), jnp.float32),
            pltpu.VMEM((BPW, DP), jnp.float32),
            pltpu.VMEM((DP,), jnp.float32),
            pltpu.SemaphoreType.DMA,
        ],
    )
    def sc_kernel(text_hbm, bias_hbm, p_hbm, out_hbm, idx_v, rows_v, res_v,
                  bias_v, sem):
        wid = lax.axis_index("s") * NC + lax.axis_index("c")
        base = wid * BPW
        pltpu.sync_copy(text_hbm.at[pl.ds(base * SEQ, BPW * SEQ)], idx_v)
        pltpu.sync_copy(bias_hbm, bias_v)
        bvec = bias_v[...]

        def group(g, _):
            gbase = g * GROWS
            handles = []
            for j in range(12):
                handles.append(pltpu.async_copy(
                    p_hbm.at[idx_v.at[pl.ds(gbase + 128 * j, 128)]],
                    rows_v.at[pl.ds(128 * j, 128)], sem))
            handles.append(pltpu.async_copy(
                p_hbm.at[idx_v.at[pl.ds(gbase + 1536, 64)]],
                rows_v.at[pl.ds(1536, 64)], sem))
            for h in handles:
                h.wait()

            zero = jnp.zeros((DP,), jnp.float32)

            def rstep(r, accs):
                return tuple(accs[k] + rows_v[k * SEQ + r] for k in range(GRP))

            accs = lax.fori_loop(0, SEQ, rstep, (zero,) * GRP)
            for k in range(GRP):
                res_v[g * GRP + k] = accs[k] + bvec
            return 0

        lax.fori_loop(0, NGRP, group, 0)
        pltpu.sync_copy(res_v, out_hbm.at[pl.ds(base, BPW)])

    return sc_kernel


_sc_kernel = _make_sc_kernel()


def kernel(text, offsets, emb, W, b):
    del offsets
    wp = jnp.zeros((D, DP), jnp.float32).at[:, :W.shape[0]].set(W.T)
    bp = jnp.zeros((DP,), jnp.float32).at[:b.shape[0]].set(b)
    p = _project(emb, wp)
    tflat = text.astype(jnp.int32).reshape(-1)
    out16 = _sc_kernel(tflat, bp, p)
    return out16[:, :W.shape[0]]


# speedup vs baseline: 15.6850x; 1.1740x over previous
import functools

import jax
import jax.numpy as jnp
from jax import lax
from jax.experimental import pallas as pl
from jax.experimental.pallas import tpu as pltpu
from jax.experimental.pallas import tpu_sc as plsc

B = 4096
SEQ = 200
D = 64
DP = 16
NC, NS = 2, 16
NW = NC * NS
BPW = B // NW
GRP = 8
NGRP = BPW // GRP
GROWS = GRP * SEQ


VPAD = 102400
SLAB = VPAD // 8
PBLK = 256


def _proj_body(e0, e1, e2, e3, e4, e5, e6, e7, wpb_ref, p_ref):
    embs = (e0, e1, e2, e3, e4, e5, e6, e7)
    acc = jnp.zeros(p_ref.shape, jnp.float32)
    for a in range(8):
        acc = acc + jnp.dot(
            embs[a][...], wpb_ref[a * D:(a + 1) * D, :],
            preferred_element_type=jnp.float32,
        )
    p_ref[...] = acc * (1.0 / SEQ)


def _project(emb, wpbig):
    v = emb.shape[0]
    nblk = v // PBLK
    grid = SLAB // PBLK

    def espec(a):
        return pl.BlockSpec(
            (PBLK, D),
            lambda j, a=a: (jnp.minimum(a * grid + j, nblk), 0),
        )

    return pl.pallas_call(
        _proj_body,
        grid=(grid,),
        in_specs=[espec(a) for a in range(8)]
        + [pl.BlockSpec((8 * D, 8 * DP), lambda j: (0, 0))],
        out_specs=pl.BlockSpec((PBLK, 8 * DP), lambda j: (j, 0)),
        out_shape=jax.ShapeDtypeStruct((SLAB, 8 * DP), jnp.float32),
    )(*([emb] * 8), wpbig)


def _make_sc_kernel():
    mesh = plsc.VectorSubcoreMesh(core_axis_name="c", subcore_axis_name="s")

    @functools.partial(
        pl.kernel,
        mesh=mesh,
        compiler_params=pltpu.CompilerParams(use_tc_tiling_on_sc=False),
        out_type=jax.ShapeDtypeStruct((B, DP), jnp.float32),
        scratch_types=[
            pltpu.VMEM((BPW * SEQ,), jnp.int32),
            pltpu.VMEM((GROWS, DP), jnp.float32),
            pltpu.VMEM((BPW, DP), jnp.float32),
            pltpu.VMEM((DP,), jnp.float32),
            pltpu.SemaphoreType.DMA,
        ],
    )
    def sc_kernel(text_hbm, bias_hbm, p_hbm, out_hbm, idx_v, rows_v, res_v,
                  bias_v, sem):
        wid = lax.axis_index("s") * NC + lax.axis_index("c")
        base = wid * BPW
        pltpu.sync_copy(text_hbm.at[pl.ds(base * SEQ, BPW * SEQ)], idx_v)
        pltpu.sync_copy(bias_hbm, bias_v)
        bvec = bias_v[...]

        def group(g, _):
            gbase = g * GROWS
            handles = []
            for j in range(12):
                handles.append(pltpu.async_copy(
                    p_hbm.at[idx_v.at[pl.ds(gbase + 128 * j, 128)]],
                    rows_v.at[pl.ds(128 * j, 128)], sem))
            handles.append(pltpu.async_copy(
                p_hbm.at[idx_v.at[pl.ds(gbase + 1536, 64)]],
                rows_v.at[pl.ds(1536, 64)], sem))
            for h in handles:
                h.wait()

            zero = jnp.zeros((DP,), jnp.float32)

            def rstep(r, accs):
                return tuple(accs[k] + rows_v[k * SEQ + r] for k in range(GRP))

            accs = lax.fori_loop(0, SEQ, rstep, (zero,) * GRP)
            for k in range(GRP):
                res_v[g * GRP + k] = accs[k] + bvec
            return 0

        lax.fori_loop(0, NGRP, group, 0)
        pltpu.sync_copy(res_v, out_hbm.at[pl.ds(base, BPW)])

    return sc_kernel


_sc_kernel = _make_sc_kernel()


def kernel(text, offsets, emb, W, b):
    del offsets
    nc = W.shape[0]
    wpbig = jnp.zeros((8 * D, 8 * DP), jnp.float32)
    for a in range(8):
        wpbig = wpbig.at[a * D:(a + 1) * D, a * DP:a * DP + nc].set(W.T)
    bp = jnp.zeros((DP,), jnp.float32).at[:nc].set(b)
    p = _project(emb, wpbig).reshape(VPAD, DP)
    t = text.astype(jnp.int32)
    tflat = ((t % SLAB) * 8 + t // SLAB).reshape(-1)
    out16 = _sc_kernel(tflat, bp, p)
    return out16[:, :nc]


# speedup vs baseline: 17.6194x; 1.1233x over previous
import functools

import jax
import jax.numpy as jnp
from jax import lax
from jax.experimental import pallas as pl
from jax.experimental.pallas import tpu as pltpu
from jax.experimental.pallas import tpu_sc as plsc

B = 4096
SEQ = 200
D = 64
DP = 16
NC, NS = 2, 16
NW = NC * NS
BPW = B // NW
GRP = 8
NGRP = BPW // GRP
GROWS = GRP * SEQ


VPAD = 102400
SLAB = VPAD // 8
PBLK = 256


def _proj_body(e0, e1, e2, e3, e4, e5, e6, e7, wpb_ref, p_ref):
    embs = (e0, e1, e2, e3, e4, e5, e6, e7)
    acc = jnp.zeros(p_ref.shape, jnp.float32)
    for a in range(8):
        acc = acc + jnp.dot(
            embs[a][...], wpb_ref[a * D:(a + 1) * D, :],
            preferred_element_type=jnp.float32,
        )
    p_ref[...] = acc * (1.0 / SEQ)


def _project(emb, wpbig):
    v = emb.shape[0]
    nblk = v // PBLK
    grid = SLAB // PBLK

    def espec(a):
        return pl.BlockSpec(
            (PBLK, D),
            lambda j, a=a: (jnp.minimum(a * grid + j, nblk), 0),
        )

    return pl.pallas_call(
        _proj_body,
        grid=(grid,),
        in_specs=[espec(a) for a in range(8)]
        + [pl.BlockSpec((8 * D, 8 * DP), lambda j: (0, 0))],
        out_specs=pl.BlockSpec((PBLK, 8 * DP), lambda j: (j, 0)),
        out_shape=jax.ShapeDtypeStruct((SLAB, 8 * DP), jnp.float32),
    )(*([emb] * 8), wpbig)


def _make_sc_kernel():
    mesh = plsc.VectorSubcoreMesh(core_axis_name="c", subcore_axis_name="s")

    @functools.partial(
        pl.kernel,
        mesh=mesh,
        compiler_params=pltpu.CompilerParams(use_tc_tiling_on_sc=False),
        out_type=jax.ShapeDtypeStruct((B, DP), jnp.float32),
        scratch_types=[
            pltpu.VMEM((BPW * SEQ,), jnp.int32),
            pltpu.VMEM((2, GROWS, DP), jnp.float32),
            pltpu.VMEM((BPW, DP), jnp.float32),
            pltpu.VMEM((DP,), jnp.float32),
            pltpu.SemaphoreType.DMA,
            pltpu.SemaphoreType.DMA,
        ],
    )
    def sc_kernel(text_hbm, bias_hbm, p_hbm, out_hbm, idx_v, rows_v, res_v,
                  bias_v, sem0, sem1):
        sems = (sem0, sem1)
        wid = lax.axis_index("s") * NC + lax.axis_index("c")
        base = wid * BPW
        pltpu.sync_copy(text_hbm.at[pl.ds(base * SEQ, BPW * SEQ)], idx_v)
        pltpu.sync_copy(bias_hbm, bias_v)
        bvec = bias_v[...]

        def fire(g, buf):
            gbase = g * GROWS
            for j in range(12):
                pltpu.async_copy(
                    p_hbm.at[idx_v.at[pl.ds(gbase + 128 * j, 128)]],
                    rows_v.at[buf, pl.ds(128 * j, 128)], sems[buf])
            pltpu.async_copy(
                p_hbm.at[idx_v.at[pl.ds(gbase + 1536, 64)]],
                rows_v.at[buf, pl.ds(1536, 64)], sems[buf])

        def drain(buf):
            pltpu.make_async_copy(
                p_hbm.at[pl.ds(0, GROWS)], rows_v.at[buf], sems[buf]).wait()

        def reduce_group(g, buf):
            zero = jnp.zeros((DP,), jnp.float32)

            def rstep(r, accs):
                return tuple(
                    accs[k] + rows_v[buf, k * SEQ + r] for k in range(GRP))

            accs = lax.fori_loop(0, SEQ, rstep, (zero,) * GRP)
            for k in range(GRP):
                res_v[g * GRP + k] = accs[k] + bvec

        fire(0, 0)
        fire(1, 1)

        def pair(i, _):
            for buf in range(2):
                g = 2 * i + buf
                drain(buf)
                reduce_group(g, buf)

                @pl.when(g + 2 < NGRP)
                def _():
                    fire(g + 2, buf)
            return 0

        lax.fori_loop(0, NGRP // 2, pair, 0)
        pltpu.sync_copy(res_v, out_hbm.at[pl.ds(base, BPW)])

    return sc_kernel


_sc_kernel = _make_sc_kernel()


def kernel(text, offsets, emb, W, b):
    del offsets
    nc = W.shape[0]
    wpbig = jnp.zeros((8 * D, 8 * DP), jnp.float32)
    for a in range(8):
        wpbig = wpbig.at[a * D:(a + 1) * D, a * DP:a * DP + nc].set(W.T)
    bp = jnp.zeros((DP,), jnp.float32).at[:nc].set(b)
    p = _project(emb, wpbig).reshape(VPAD, DP)
    t = text.astype(jnp.int32)
    tflat = ((t % SLAB) * 8 + t // SLAB).reshape(-1)
    out16 = _sc_kernel(tflat, bp, p)
    return out16[:, :nc]


# speedup vs baseline: 19.4309x; 1.1028x over previous
import functools

import jax
import jax.numpy as jnp
from jax import lax
from jax.experimental import pallas as pl
from jax.experimental.pallas import tpu as pltpu
from jax.experimental.pallas import tpu_sc as plsc

B = 4096
SEQ = 200
D = 64
DP = 16
NC, NS = 2, 16
NW = NC * NS
BPW = B // NW
GRP = 8
NGRP = BPW // GRP
GROWS = GRP * SEQ


VPAD = 102400
SLAB = VPAD // 8
PBLK = 512


def _proj_body(e0, e1, e2, e3, e4, e5, e6, e7, wpb_ref, p_ref):
    embs = (e0, e1, e2, e3, e4, e5, e6, e7)
    acc = jnp.zeros(p_ref.shape, jnp.float32)
    for a in range(8):
        acc = acc + jnp.dot(
            embs[a][...], wpb_ref[a * D:(a + 1) * D, :],
            preferred_element_type=jnp.float32,
        )
    p_ref[...] = acc * (1.0 / SEQ)


def _project(emb, wpbig):
    v = emb.shape[0]
    nblk = v // PBLK
    grid = SLAB // PBLK

    def espec(a):
        return pl.BlockSpec(
            (PBLK, D),
            lambda j, a=a: (jnp.minimum(a * grid + j, nblk), 0),
        )

    return pl.pallas_call(
        _proj_body,
        grid=(grid,),
        in_specs=[espec(a) for a in range(8)]
        + [pl.BlockSpec((8 * D, 8 * DP), lambda j: (0, 0))],
        out_specs=pl.BlockSpec((PBLK, 8 * DP), lambda j: (j, 0)),
        out_shape=jax.ShapeDtypeStruct((SLAB, 8 * DP), jnp.float32),
    )(*([emb] * 8), wpbig)


def _make_sc_kernel():
    mesh = plsc.VectorSubcoreMesh(core_axis_name="c", subcore_axis_name="s")

    @functools.partial(
        pl.kernel,
        mesh=mesh,
        compiler_params=pltpu.CompilerParams(use_tc_tiling_on_sc=False),
        out_type=jax.ShapeDtypeStruct((B, DP), jnp.float32),
        scratch_types=[
            pltpu.VMEM((BPW * SEQ,), jnp.int32),
            pltpu.VMEM((2, GROWS, DP), jnp.float32),
            pltpu.VMEM((BPW, DP), jnp.float32),
            pltpu.VMEM((DP,), jnp.float32),
            pltpu.SemaphoreType.DMA,
            pltpu.SemaphoreType.DMA,
        ],
    )
    def sc_kernel(text_hbm, bias_hbm, p_hbm, out_hbm, idx_v, rows_v, res_v,
                  bias_v, sem0, sem1):
        sems = (sem0, sem1)
        wid = lax.axis_index("s") * NC + lax.axis_index("c")
        base = wid * BPW
        pltpu.sync_copy(text_hbm.at[pl.ds(base * SEQ, BPW * SEQ)], idx_v)
        pltpu.sync_copy(bias_hbm, bias_v)
        bvec = bias_v[...]

        def fire(g, buf):
            gbase = g * GROWS
            for j in range(12):
                pltpu.async_copy(
                    p_hbm.at[idx_v.at[pl.ds(gbase + 128 * j, 128)]],
                    rows_v.at[buf, pl.ds(128 * j, 128)], sems[buf])
            pltpu.async_copy(
                p_hbm.at[idx_v.at[pl.ds(gbase + 1536, 64)]],
                rows_v.at[buf, pl.ds(1536, 64)], sems[buf])

        def drain(buf):
            pltpu.make_async_copy(
                p_hbm.at[pl.ds(0, GROWS)], rows_v.at[buf], sems[buf]).wait()

        def reduce_group(g, buf):
            zero = jnp.zeros((DP,), jnp.float32)

            def rstep(r, accs):
                return tuple(
                    accs[k] + rows_v[buf, k * SEQ + r] for k in range(GRP))

            accs = lax.fori_loop(0, SEQ, rstep, (zero,) * GRP)
            for k in range(GRP):
                res_v[g * GRP + k] = accs[k] + bvec

        fire(0, 0)
        fire(1, 1)

        def pair(i, _):
            for buf in range(2):
                g = 2 * i + buf
                drain(buf)
                reduce_group(g, buf)

                @pl.when(g + 2 < NGRP)
                def _():
                    fire(g + 2, buf)
            return 0

        lax.fori_loop(0, NGRP // 2, pair, 0)
        pltpu.sync_copy(res_v, out_hbm.at[pl.ds(base, BPW)])

    return sc_kernel


_sc_kernel = _make_sc_kernel()


def kernel(text, offsets, emb, W, b):
    del offsets
    nc = W.shape[0]
    wpbig = jnp.zeros((8 * D, 8 * DP), jnp.float32)
    for a in range(8):
        wpbig = wpbig.at[a * D:(a + 1) * D, a * DP:a * DP + nc].set(W.T)
    bp = jnp.zeros((DP,), jnp.float32).at[:nc].set(b)
    p = _project(emb, wpbig).reshape(VPAD, DP)
    t = text.astype(jnp.int32)
    tflat = ((t % SLAB) * 8 + t // SLAB).reshape(-1)
    out16 = _sc_kernel(tflat, bp, p)
    return out16[:, :nc]


# speedup vs baseline: 20.6564x; 1.0631x over previous
import functools

import jax
import jax.numpy as jnp
from jax import lax
from jax.experimental import pallas as pl
from jax.experimental.pallas import tpu as pltpu
from jax.experimental.pallas import tpu_sc as plsc

B = 4096
SEQ = 200
D = 64
DP = 16
NC, NS = 2, 16
NW = NC * NS
BPW = B // NW
GRP = 8
NGRP = BPW // GRP
GROWS = GRP * SEQ


VPAD = 102400
SLAB = VPAD // 8
PBLK = 1280


def _proj_body(e0, e1, e2, e3, e4, e5, e6, e7, wpb_ref, p_ref):
    embs = (e0, e1, e2, e3, e4, e5, e6, e7)
    acc = jnp.zeros(p_ref.shape, jnp.float32)
    for a in range(8):
        acc = acc + jnp.dot(
            embs[a][...], wpb_ref[a * D:(a + 1) * D, :],
            preferred_element_type=jnp.float32,
        )
    p_ref[...] = acc * (1.0 / SEQ)


def _project(emb, wpbig):
    v = emb.shape[0]
    nblk = v // PBLK
    grid = SLAB // PBLK

    def espec(a):
        return pl.BlockSpec(
            (PBLK, D),
            lambda j, a=a: (jnp.minimum(a * grid + j, nblk), 0),
        )

    return pl.pallas_call(
        _proj_body,
        grid=(grid,),
        in_specs=[espec(a) for a in range(8)]
        + [pl.BlockSpec((8 * D, 8 * DP), lambda j: (0, 0))],
        out_specs=pl.BlockSpec((PBLK, 8 * DP), lambda j: (j, 0)),
        out_shape=jax.ShapeDtypeStruct((SLAB, 8 * DP), jnp.float32),
    )(*([emb] * 8), wpbig)


def _make_sc_kernel():
    mesh = plsc.VectorSubcoreMesh(core_axis_name="c", subcore_axis_name="s")

    @functools.partial(
        pl.kernel,
        mesh=mesh,
        compiler_params=pltpu.CompilerParams(use_tc_tiling_on_sc=False),
        out_type=jax.ShapeDtypeStruct((B, DP), jnp.float32),
        scratch_types=[
            pltpu.VMEM((BPW * SEQ,), jnp.int32),
            pltpu.VMEM((2, GROWS, DP), jnp.float32),
            pltpu.VMEM((BPW, DP), jnp.float32),
            pltpu.VMEM((DP,), jnp.float32),
            pltpu.SemaphoreType.DMA,
            pltpu.SemaphoreType.DMA,
        ],
    )
    def sc_kernel(text_hbm, bias_hbm, p_hbm, out_hbm, idx_v, rows_v, res_v,
                  bias_v, sem0, sem1):
        sems = (sem0, sem1)
        wid = lax.axis_index("s") * NC + lax.axis_index("c")
        base = wid * BPW
        pltpu.sync_copy(text_hbm.at[pl.ds(base * SEQ, BPW * SEQ)], idx_v)
        pltpu.sync_copy(bias_hbm, bias_v)
        bvec = bias_v[...]

        def fire(g, buf):
            gbase = g * GROWS
            for j in range(12):
                pltpu.async_copy(
                    p_hbm.at[idx_v.at[pl.ds(gbase + 128 * j, 128)]],
                    rows_v.at[buf, pl.ds(128 * j, 128)], sems[buf])
            pltpu.async_copy(
                p_hbm.at[idx_v.at[pl.ds(gbase + 1536, 64)]],
                rows_v.at[buf, pl.ds(1536, 64)], sems[buf])

        def drain(buf):
            pltpu.make_async_copy(
                p_hbm.at[pl.ds(0, GROWS)], rows_v.at[buf], sems[buf]).wait()

        def reduce_group(g, buf):
            zero = jnp.zeros((DP,), jnp.float32)

            def rstep(r, accs):
                return tuple(
                    accs[k] + rows_v[buf, k * SEQ + r] for k in range(GRP))

            accs = lax.fori_loop(0, SEQ, rstep, (zero,) * GRP)
            for k in range(GRP):
                res_v[g * GRP + k] = accs[k] + bvec

        fire(0, 0)
        fire(1, 1)

        def pair(i, _):
            for buf in range(2):
                g = 2 * i + buf
                drain(buf)
                reduce_group(g, buf)

                @pl.when(g + 2 < NGRP)
                def _():
                    fire(g + 2, buf)
            return 0

        lax.fori_loop(0, NGRP // 2, pair, 0)
        pltpu.sync_copy(res_v, out_hbm.at[pl.ds(base, BPW)])

    return sc_kernel


_sc_kernel = _make_sc_kernel()


def kernel(text, offsets, emb, W, b):
    del offsets
    nc = W.shape[0]
    wpbig = jnp.zeros((8 * D, 8 * DP), jnp.float32)
    for a in range(8):
        wpbig = wpbig.at[a * D:(a + 1) * D, a * DP:a * DP + nc].set(W.T)
    bp = jnp.zeros((DP,), jnp.float32).at[:nc].set(b)
    p = _project(emb, wpbig).reshape(VPAD, DP)
    t = text.astype(jnp.int32)
    tflat = ((t % SLAB) * 8 + t // SLAB).reshape(-1)
    out16 = _sc_kernel(tflat, bp, p)
    return out16[:, :nc]


# speedup vs baseline: 27.2857x; 1.3209x over previous
import functools

import jax
import jax.numpy as jnp
from jax import lax
from jax.experimental import pallas as pl
from jax.experimental.pallas import tpu as pltpu
from jax.experimental.pallas import tpu_sc as plsc

B = 4096
SEQ = 200
D = 64
DP = 16
NC, NS = 2, 16
NW = NC * NS
BPW = B // NW
GRP = 8
NGRP = BPW // GRP
GROWS = GRP * SEQ


VPAD = 102400
SLAB = VPAD // 8
PBLK = 1280


def _proj_body(e0, e1, e2, e3, e4, e5, e6, e7, wpb_ref, p_ref):
    embs = (e0, e1, e2, e3, e4, e5, e6, e7)
    acc = jnp.zeros(p_ref.shape, jnp.float32)
    for a in range(8):
        acc = acc + lax.dot_general(
            embs[a][...], wpb_ref[a * D:(a + 1) * D, :],
            (((0,), (0,)), ((), ())),
            preferred_element_type=jnp.float32,
        )
    p_ref[...] = acc * (1.0 / SEQ)


def _project(embt, wpbig):
    v = embt.shape[1]
    nblk = v // PBLK
    grid = SLAB // PBLK

    def espec(a):
        return pl.BlockSpec(
            (D, PBLK),
            lambda j, a=a: (0, jnp.minimum(a * grid + j, nblk)),
        )

    return pl.pallas_call(
        _proj_body,
        grid=(grid,),
        in_specs=[espec(a) for a in range(8)]
        + [pl.BlockSpec((8 * D, 8 * DP), lambda j: (0, 0))],
        out_specs=pl.BlockSpec((PBLK, 8 * DP), lambda j: (j, 0)),
        out_shape=jax.ShapeDtypeStruct((SLAB, 8 * DP), jnp.float32),
    )(*([embt] * 8), wpbig)


def _make_sc_kernel():
    mesh = plsc.VectorSubcoreMesh(core_axis_name="c", subcore_axis_name="s")

    @functools.partial(
        pl.kernel,
        mesh=mesh,
        compiler_params=pltpu.CompilerParams(use_tc_tiling_on_sc=False),
        out_type=jax.ShapeDtypeStruct((B, DP), jnp.float32),
        scratch_types=[
            pltpu.VMEM((BPW * SEQ,), jnp.int32),
            pltpu.VMEM((2, GROWS, DP), jnp.float32),
            pltpu.VMEM((BPW, DP), jnp.float32),
            pltpu.VMEM((DP,), jnp.float32),
            pltpu.SemaphoreType.DMA,
            pltpu.SemaphoreType.DMA,
        ],
    )
    def sc_kernel(text_hbm, bias_hbm, p_hbm, out_hbm, idx_v, rows_v, res_v,
                  bias_v, sem0, sem1):
        sems = (sem0, sem1)
        wid = lax.axis_index("s") * NC + lax.axis_index("c")
        base = wid * BPW
        pltpu.sync_copy(text_hbm.at[pl.ds(base * SEQ, BPW * SEQ)], idx_v)
        pltpu.sync_copy(bias_hbm, bias_v)
        bvec = bias_v[...]

        def fire(g, buf):
            gbase = g * GROWS
            for j in range(12):
                pltpu.async_copy(
                    p_hbm.at[idx_v.at[pl.ds(gbase + 128 * j, 128)]],
                    rows_v.at[buf, pl.ds(128 * j, 128)], sems[buf])
            pltpu.async_copy(
                p_hbm.at[idx_v.at[pl.ds(gbase + 1536, 64)]],
                rows_v.at[buf, pl.ds(1536, 64)], sems[buf])

        def drain(buf):
            pltpu.make_async_copy(
                p_hbm.at[pl.ds(0, GROWS)], rows_v.at[buf], sems[buf]).wait()

        def reduce_group(g, buf):
            zero = jnp.zeros((DP,), jnp.float32)

            def rstep(r, accs):
                return tuple(
                    accs[k] + rows_v[buf, k * SEQ + r] for k in range(GRP))

            accs = lax.fori_loop(0, SEQ, rstep, (zero,) * GRP)
            for k in range(GRP):
                res_v[g * GRP + k] = accs[k] + bvec

        fire(0, 0)
        fire(1, 1)

        def pair(i, _):
            for buf in range(2):
                g = 2 * i + buf
                drain(buf)
                reduce_group(g, buf)

                @pl.when(g + 2 < NGRP)
                def _():
                    fire(g + 2, buf)
            return 0

        lax.fori_loop(0, NGRP // 2, pair, 0)
        pltpu.sync_copy(res_v, out_hbm.at[pl.ds(base, BPW)])

    return sc_kernel


_sc_kernel = _make_sc_kernel()


def kernel(text, offsets, emb, W, b):
    del offsets
    nc = W.shape[0]
    wpbig = jnp.zeros((8 * D, 8 * DP), jnp.float32)
    for a in range(8):
        wpbig = wpbig.at[a * D:(a + 1) * D, a * DP:a * DP + nc].set(W.T)
    bp = jnp.zeros((DP,), jnp.float32).at[:nc].set(b)
    p = _project(emb.T, wpbig).reshape(VPAD, DP)
    t = text.astype(jnp.int32)
    tflat = ((t % SLAB) * 8 + t // SLAB).reshape(-1)
    out16 = _sc_kernel(tflat, bp, p)
    return out16[:, :nc]


# speedup vs baseline: 27.9173x; 1.0231x over previous
import functools

import jax
import jax.numpy as jnp
from jax import lax
from jax.experimental import pallas as pl
from jax.experimental.pallas import tpu as pltpu
from jax.experimental.pallas import tpu_sc as plsc

B = 4096
SEQ = 200
D = 64
DP = 16
NC, NS = 2, 16
NW = NC * NS
BPW = B // NW
GRP = 8
NGRP = BPW // GRP
GROWS = GRP * SEQ


VPAD = 131072
SLAB = VPAD // 8
PBLK = 1024


def _proj_body(e0, e1, e2, e3, e4, e5, e6, e7, wpb_ref, p_ref):
    embs = (e0, e1, e2, e3, e4, e5, e6, e7)
    acc = jnp.zeros(p_ref.shape, jnp.float32)
    for a in range(8):
        acc = acc + lax.dot_general(
            embs[a][...], wpb_ref[a * D:(a + 1) * D, :],
            (((0,), (0,)), ((), ())),
            preferred_element_type=jnp.float32,
        )
    p_ref[...] = acc * (1.0 / SEQ)


def _project(embt, wpbig):
    v = embt.shape[1]
    nblk = v // PBLK
    grid = SLAB // PBLK

    def espec(a):
        return pl.BlockSpec(
            (D, PBLK),
            lambda j, a=a: (0, jnp.minimum(a * grid + j, nblk)),
        )

    return pl.pallas_call(
        _proj_body,
        grid=(grid,),
        in_specs=[espec(a) for a in range(8)]
        + [pl.BlockSpec((8 * D, 8 * DP), lambda j: (0, 0))],
        out_specs=pl.BlockSpec((PBLK, 8 * DP), lambda j: (j, 0)),
        out_shape=jax.ShapeDtypeStruct((SLAB, 8 * DP), jnp.float32),
    )(*([embt] * 8), wpbig)


def _make_sc_kernel():
    mesh = plsc.VectorSubcoreMesh(core_axis_name="c", subcore_axis_name="s")

    @functools.partial(
        pl.kernel,
        mesh=mesh,
        compiler_params=pltpu.CompilerParams(use_tc_tiling_on_sc=False),
        out_type=jax.ShapeDtypeStruct((B, DP), jnp.float32),
        scratch_types=[
            pltpu.VMEM((BPW * SEQ,), jnp.int32),
            pltpu.VMEM((2, GROWS, DP), jnp.float32),
            pltpu.VMEM((BPW, DP), jnp.float32),
            pltpu.VMEM((DP,), jnp.float32),
            pltpu.SemaphoreType.DMA,
            pltpu.SemaphoreType.DMA,
        ],
    )
    def sc_kernel(text_hbm, bias_hbm, p_hbm, out_hbm, idx_v, rows_v, res_v,
                  bias_v, sem0, sem1):
        sems = (sem0, sem1)
        wid = lax.axis_index("s") * NC + lax.axis_index("c")
        base = wid * BPW
        pltpu.sync_copy(text_hbm.at[pl.ds(base * SEQ, BPW * SEQ)], idx_v)
        pltpu.sync_copy(bias_hbm, bias_v)
        bvec = bias_v[...]

        def fire(g, buf):
            gbase = g * GROWS
            for j in range(12):
                pltpu.async_copy(
                    p_hbm.at[idx_v.at[pl.ds(gbase + 128 * j, 128)]],
                    rows_v.at[buf, pl.ds(128 * j, 128)], sems[buf])
            pltpu.async_copy(
                p_hbm.at[idx_v.at[pl.ds(gbase + 1536, 64)]],
                rows_v.at[buf, pl.ds(1536, 64)], sems[buf])

        def drain(buf):
            pltpu.make_async_copy(
                p_hbm.at[pl.ds(0, GROWS)], rows_v.at[buf], sems[buf]).wait()

        def reduce_group(g, buf):
            zero = jnp.zeros((DP,), jnp.float32)

            def rstep(r, accs):
                return tuple(
                    accs[k] + rows_v[buf, k * SEQ + r] for k in range(GRP))

            accs = lax.fori_loop(0, SEQ, rstep, (zero,) * GRP)
            for k in range(GRP):
                res_v[g * GRP + k] = accs[k] + bvec

        fire(0, 0)
        fire(1, 1)

        def pair(i, _):
            for buf in range(2):
                g = 2 * i + buf
                drain(buf)
                reduce_group(g, buf)

                @pl.when(g + 2 < NGRP)
                def _():
                    fire(g + 2, buf)
            return 0

        lax.fori_loop(0, NGRP // 2, pair, 0)
        pltpu.sync_copy(res_v, out_hbm.at[pl.ds(base, BPW)])

    return sc_kernel


_sc_kernel = _make_sc_kernel()


def kernel(text, offsets, emb, W, b):
    del offsets
    nc = W.shape[0]
    wp = jnp.zeros((D, DP), jnp.float32).at[:, :nc].set(W.T)
    wpbig = jnp.kron(jnp.eye(8, dtype=jnp.float32), wp)
    bp = jnp.zeros((DP,), jnp.float32).at[:nc].set(b)
    p = _project(emb.T, wpbig).reshape(VPAD, DP)
    t = text.astype(jnp.int32)
    tflat = ((t & (SLAB - 1)) * 8 + (t >> 14)).reshape(-1)
    out16 = _sc_kernel(tflat, bp, p)
    return out16[:, :nc]


# speedup vs baseline: 28.5429x; 1.0224x over previous
import functools

import jax
import jax.numpy as jnp
from jax import lax
from jax.experimental import pallas as pl
from jax.experimental.pallas import tpu as pltpu
from jax.experimental.pallas import tpu_sc as plsc

B = 4096
SEQ = 200
D = 64
DP = 16
NC, NS = 2, 16
NW = NC * NS
BPW = B // NW
GRP = 8
NGRP = BPW // GRP
GROWS = GRP * SEQ


VPAD = 131072
SLAB = VPAD // 8
PBLK = 1024


def _proj_body(e0, e1, e2, e3, e4, e5, e6, e7, wpb_ref, p_ref):
    embs = (e0, e1, e2, e3, e4, e5, e6, e7)
    acc = jnp.zeros(p_ref.shape, jnp.float32)
    for a in range(8):
        acc = acc + lax.dot_general(
            embs[a][...], wpb_ref[a * D:(a + 1) * D, :],
            (((0,), (0,)), ((), ())),
            preferred_element_type=jnp.float32,
        )
    p_ref[...] = acc * (1.0 / SEQ)


def _project(embt, wpbig):
    v = embt.shape[1]
    nblk = v // PBLK
    grid = SLAB // PBLK

    def espec(a):
        return pl.BlockSpec(
            (D, PBLK),
            lambda j, a=a: (0, jnp.minimum(a * grid + j, nblk)),
        )

    return pl.pallas_call(
        _proj_body,
        grid=(grid,),
        in_specs=[espec(a) for a in range(8)]
        + [pl.BlockSpec((8 * D, 8 * DP), lambda j: (0, 0))],
        out_specs=pl.BlockSpec((PBLK, 8 * DP), lambda j: (j, 0)),
        out_shape=jax.ShapeDtypeStruct((SLAB, 8 * DP), jnp.float32),
    )(*([embt] * 8), wpbig)


def _make_sc_kernel():
    mesh = plsc.VectorSubcoreMesh(core_axis_name="c", subcore_axis_name="s")

    @functools.partial(
        pl.kernel,
        mesh=mesh,
        compiler_params=pltpu.CompilerParams(
            use_tc_tiling_on_sc=False, needs_layout_passes=False),
        out_type=jax.ShapeDtypeStruct((B, DP), jnp.float32),
        scratch_types=[
            pltpu.VMEM((SEQ, BPW), jnp.int32),
            pltpu.VMEM((BPW * SEQ,), jnp.int32),
            pltpu.VMEM((2, GROWS, DP), jnp.float32),
            pltpu.VMEM((BPW, DP), jnp.float32),
            pltpu.VMEM((DP,), jnp.float32),
            pltpu.SemaphoreType.DMA,
            pltpu.SemaphoreType.DMA,
        ],
    )
    def sc_kernel(textt_hbm, bias_hbm, p_hbm, out_hbm, stage_v, idx_v, rows_v,
                  res_v, bias_v, sem0, sem1):
        sems = (sem0, sem1)
        wid = lax.axis_index("s") * NC + lax.axis_index("c")
        base = wid * BPW
        pltpu.sync_copy(textt_hbm.at[:, pl.ds(base, BPW)], stage_v)
        pltpu.sync_copy(bias_hbm, bias_v)
        bvec = bias_v[...]
        lanes = lax.iota(jnp.int32, 16) * SEQ

        def transpose_step(t, _):
            for k in range(BPW // 16):
                x = stage_v[t, pl.ds(16 * k, 16)]
                r = ((x & (SLAB - 1)) << 3) | lax.shift_right_logical(x, 14)
                plsc.store_scatter(idx_v, [lanes + (t + 16 * k * SEQ)], r)
            return 0

        lax.fori_loop(0, SEQ, transpose_step, 0)

        def fire(g, buf):
            gbase = g * GROWS
            for j in range(12):
                pltpu.async_copy(
                    p_hbm.at[idx_v.at[pl.ds(gbase + 128 * j, 128)]],
                    rows_v.at[buf, pl.ds(128 * j, 128)], sems[buf])
            pltpu.async_copy(
                p_hbm.at[idx_v.at[pl.ds(gbase + 1536, 64)]],
                rows_v.at[buf, pl.ds(1536, 64)], sems[buf])

        def drain(buf):
            pltpu.make_async_copy(
                p_hbm.at[pl.ds(0, GROWS)], rows_v.at[buf], sems[buf]).wait()

        def reduce_group(g, buf):
            zero = jnp.zeros((DP,), jnp.float32)

            def rstep(r, accs):
                return tuple(
                    accs[k] + rows_v[buf, k * SEQ + r] for k in range(GRP))

            accs = lax.fori_loop(0, SEQ, rstep, (zero,) * GRP)
            for k in range(GRP):
                res_v[g * GRP + k] = accs[k] + bvec

        fire(0, 0)
        fire(1, 1)

        def pair(i, _):
            for buf in range(2):
                g = 2 * i + buf
                drain(buf)
                reduce_group(g, buf)

                @pl.when(g + 2 < NGRP)
                def _():
                    fire(g + 2, buf)
            return 0

        lax.fori_loop(0, NGRP // 2, pair, 0)
        pltpu.sync_copy(res_v, out_hbm.at[pl.ds(base, BPW)])

    return sc_kernel


_sc_kernel = _make_sc_kernel()


def kernel(text, offsets, emb, W, b):
    del offsets
    nc = W.shape[0]
    wp = jnp.zeros((D, DP), jnp.float32).at[:, :nc].set(W.T)
    wpbig = jnp.kron(jnp.eye(8, dtype=jnp.float32), wp)
    bp = jnp.zeros((DP,), jnp.float32).at[:nc].set(b)
    p = _project(emb.T, wpbig).reshape(VPAD, DP)
    out16 = _sc_kernel(text.astype(jnp.int32).T, bp, p)
    return out16[:, :nc]


# speedup vs baseline: 32.7839x; 1.1486x over previous
import functools

import jax
import jax.numpy as jnp
from jax import lax
from jax.experimental import pallas as pl
from jax.experimental.pallas import tpu as pltpu
from jax.experimental.pallas import tpu_sc as plsc

B = 4096
SEQ = 200
D = 64
DP = 16
NC, NS = 2, 16
NW = NC * NS
BPW = B // NW
GRP = 8
NGRP = BPW // GRP
GROWS = GRP * SEQ


VPAD = 102400
SLAB = VPAD // 8
PBLK = 1280


def _proj_body(e0, e1, e2, e3, e4, e5, e6, e7, wpb_ref, p_ref):
    embs = (e0, e1, e2, e3, e4, e5, e6, e7)
    acc = jnp.zeros(p_ref.shape, jnp.float32)
    for a in range(8):
        acc = acc + lax.dot_general(
            embs[a][...], wpb_ref[a * D:(a + 1) * D, :],
            (((0,), (0,)), ((), ())),
            preferred_element_type=jnp.float32,
        )
    p_ref[...] = acc * (1.0 / SEQ)


def _project(embt, wpbig):
    v = embt.shape[1]
    nblk = v // PBLK
    grid = SLAB // PBLK

    def espec(a):
        return pl.BlockSpec(
            (D, PBLK),
            lambda j, a=a: (0, jnp.minimum(a * grid + j, nblk)),
        )

    return pl.pallas_call(
        _proj_body,
        grid=(grid,),
        in_specs=[espec(a) for a in range(8)]
        + [pl.BlockSpec((8 * D, 8 * DP), lambda j: (0, 0))],
        out_specs=pl.BlockSpec((PBLK, 8 * DP), lambda j: (j, 0)),
        out_shape=jax.ShapeDtypeStruct((SLAB, 8 * DP), jnp.float32),
    )(*([embt] * 8), wpbig)


def _make_sc_kernel():
    mesh = plsc.VectorSubcoreMesh(core_axis_name="c", subcore_axis_name="s")

    @functools.partial(
        pl.kernel,
        mesh=mesh,
        compiler_params=pltpu.CompilerParams(
            use_tc_tiling_on_sc=False, needs_layout_passes=False),
        out_type=jax.ShapeDtypeStruct((B, DP), jnp.float32),
        scratch_types=[
            pltpu.VMEM((SEQ, BPW), jnp.int32),
            pltpu.VMEM((BPW * SEQ,), jnp.int32),
            pltpu.VMEM((2, GROWS, DP), jnp.float32),
            pltpu.VMEM((BPW, DP), jnp.float32),
            pltpu.VMEM((DP,), jnp.float32),
            pltpu.SemaphoreType.DMA,
            pltpu.SemaphoreType.DMA,
        ],
    )
    def sc_kernel(textt_hbm, bias_hbm, p_hbm, out_hbm, stage_v, idx_v, rows_v,
                  res_v, bias_v, sem0, sem1):
        sems = (sem0, sem1)
        wid = lax.axis_index("s") * NC + lax.axis_index("c")
        base = wid * BPW
        pltpu.sync_copy(textt_hbm.at[:, pl.ds(base, BPW)], stage_v)
        pltpu.sync_copy(bias_hbm, bias_v)
        bvec = bias_v[...]
        lanes = lax.iota(jnp.int32, 16) * SEQ

        def transform_band(m):
            def step(t, _):
                x = stage_v[t, pl.ds(16 * m, 16)]
                a = lax.shift_right_logical(
                    lax.shift_right_logical(x, 9) * 1311, 15)
                r = ((x - a * SLAB) << 3) | a
                plsc.store_scatter(idx_v, [lanes + (t + 16 * m * SEQ)], r)
                return 0

            lax.fori_loop(0, SEQ, step, 0)

        def fire(g, buf):
            gbase = g * GROWS
            for j in range(12):
                pltpu.async_copy(
                    p_hbm.at[idx_v.at[pl.ds(gbase + 128 * j, 128)]],
                    rows_v.at[buf, pl.ds(128 * j, 128)], sems[buf])
            pltpu.async_copy(
                p_hbm.at[idx_v.at[pl.ds(gbase + 1536, 64)]],
                rows_v.at[buf, pl.ds(1536, 64)], sems[buf])

        def drain(buf):
            pltpu.make_async_copy(
                p_hbm.at[pl.ds(0, GROWS)], rows_v.at[buf], sems[buf]).wait()

        def reduce_group(g, buf):
            zero = jnp.zeros((DP,), jnp.float32)

            def rstep(r, accs):
                return tuple(
                    accs[k] + rows_v[buf, k * SEQ + r] for k in range(GRP))

            accs = lax.fori_loop(0, SEQ, rstep, (zero,) * GRP)
            for k in range(GRP):
                res_v[g * GRP + k] = accs[k] + bvec

        transform_band(0)
        fire(0, 0)
        fire(1, 1)

        def pair(i, _):
            @pl.when(i + 1 < NGRP // 2)
            def _():
                transform_band(i + 1)

            for buf in range(2):
                g = 2 * i + buf
                drain(buf)
                reduce_group(g, buf)

                @pl.when(g + 2 < NGRP)
                def _():
                    fire(g + 2, buf)
            return 0

        lax.fori_loop(0, NGRP // 2, pair, 0)
        pltpu.sync_copy(res_v, out_hbm.at[pl.ds(base, BPW)])

    return sc_kernel


_sc_kernel = _make_sc_kernel()


def kernel(text, offsets, emb, W, b):
    del offsets
    nc = W.shape[0]
    wp = jnp.zeros((D, DP), jnp.float32).at[:, :nc].set(W.T)
    wpbig = jnp.kron(jnp.eye(8, dtype=jnp.float32), wp)
    bp = jnp.zeros((DP,), jnp.float32).at[:nc].set(b)
    p = _project(emb.T, wpbig).reshape(VPAD, DP)
    out16 = _sc_kernel(text.astype(jnp.int32).T, bp, p)
    return out16[:, :nc]
